# baseline (speedup 1.0000x reference)
import functools
import math

import jax
import jax.numpy as jnp
from jax import lax
from jax.experimental import pallas as pl
from jax.experimental.pallas import tpu as pltpu
from jax.experimental.pallas import tpu_sc as plsc

N = 262144
A = 16
K = N // 2
SEL_COLS = 2048

NW = 32
TPW = N // NW
CR = 128
NCH = TPW // CR
NG = CR // 16

_LOGC = -0.5 * A * math.log(2.0 * math.pi)


def _sortable_i32(x_f32):
    b = lax.bitcast_convert_type(x_f32, jnp.int32)
    return b ^ ((b >> 31) & jnp.int32(0x7FFFFFFF))


def _select_body(temp_ref, adv_ref, out_i_ref, out_f_ref):
    a = adv_ref[...]
    s = _sortable_i32(a)

    def vbody(t, cand):
        trial = cand + (jnp.int32(1) << (31 - t))
        c = jnp.sum((s >= trial).astype(jnp.int32))
        return lax.select(c >= K, trial, cand)

    theta = lax.fori_loop(0, 32, vbody, jnp.int32(-2147483648))

    c_gt = jnp.sum((s > theta).astype(jnp.int32))
    t_need = K - c_gt

    eq = (s == theta)
    idx = (lax.broadcasted_iota(jnp.int32, (N // SEL_COLS, SEL_COLS), 0)
           * SEL_COLS
           + lax.broadcasted_iota(jnp.int32, (N // SEL_COLS, SEL_COLS), 1))

    def ibody(t, m):
        trial = m | (jnp.int32(1) << (17 - t))
        c = jnp.sum((eq & (idx < trial)).astype(jnp.int32))
        return lax.select(c < t_need, trial, m)

    mbound = lax.fori_loop(0, 18, ibody, jnp.int32(0))

    out_i_ref[0] = theta
    out_i_ref[1] = mbound
    for t in range(2, 16):
        out_i_ref[t] = jnp.int32(0)
    out_f_ref[0] = jnp.max(a)
    out_f_ref[1] = temp_ref[0] + jnp.float32(0.001)
    for t in range(2, 16):
        out_f_ref[t] = jnp.float32(0.0)


def _sc_body(m_hbm, s_hbm, a_hbm, advg_hbm, pi_hbm, pf_hbm, out_hbm,
             mbuf, sbuf, abuf, advbuf, pibuf, pfbuf, obuf):
    nc = 2
    wid = lax.axis_index("s") * nc + lax.axis_index("c")
    base = wid * TPW

    pltpu.sync_copy(pi_hbm, pibuf)
    pltpu.sync_copy(pf_hbm, pfbuf)
    pivec = pibuf[...]
    pfvec = pfbuf[...]
    theta = pivec[0]
    mbound = pivec[1]
    mx = pfvec[0]
    invtpv = jnp.full((16,), 1.0, jnp.float32) / pfvec
    invtp = invtpv[1]

    half = jnp.float32(-0.5)
    c1 = jnp.float32(2.0)
    c3 = jnp.float32(2.0 / 3.0)
    c5 = jnp.float32(0.4)
    c7 = jnp.float32(2.0 / 7.0)

    def chunk_body(c, carry):
        nu_v, d_v = carry
        cb = base + c * CR
        cbt = pl.multiple_of(cb, CR)
        cba = pl.multiple_of(cb // 16, CR // 16)
        pltpu.sync_copy(m_hbm.at[pl.ds(cbt, CR), :], mbuf)
        pltpu.sync_copy(s_hbm.at[pl.ds(cbt, CR), :], sbuf)
        pltpu.sync_copy(a_hbm.at[pl.ds(cbt, CR), :], abuf)
        pltpu.sync_copy(advg_hbm.at[pl.ds(cba, CR // 16), :], advbuf)

        def group_body(g, carry2):
            nu_v2, d_v2 = carry2
            adv_v = advbuf[g]
            sv = _sortable_i32(adv_v)
            idx_v = cb + g * 16 + lax.iota(jnp.int32, 16)
            sel = (sv > theta) | ((sv == theta) & (idx_v <= mbound))
            w_v = jnp.where(sel, jnp.exp((adv_v - mx) * invtp),
                            jnp.float32(0.0))
            d_v2 = d_v2 + w_v
            for j in range(16):
                i = g * 16 + j
                mv = mbuf[i]
                stdv = sbuf[i]
                av = abuf[i]
                rs = jnp.full((16,), 1.0, jnp.float32) / stdv
                t = (av - mv) * rs
                wl = (rs - jnp.float32(1.0)) / (rs + jnp.float32(1.0))
                wl2 = wl * wl
                logs = -(wl * (c1 + wl2 * (c3 + wl2 * (c5 + wl2 * c7))))
                term = half * (t * t) - logs
                nu_v2 = nu_v2 + w_v[j] * term
            return (nu_v2, d_v2)

        return lax.fori_loop(0, NG, group_body, (nu_v, d_v))

    zero = jnp.zeros((16,), jnp.float32)
    nu_v, d_v = lax.fori_loop(0, NCH, chunk_body, (zero, zero))

    obuf[0] = d_v
    obuf[1] = nu_v
    pltpu.sync_copy(obuf, out_hbm.at[wid])


@jax.jit
def kernel(action_mean, action_std, actions, temperature, advantages):
    advS = advantages.reshape(N // SEL_COLS, SEL_COLS)
    advG = advantages.reshape(N // 16, 16)

    sel_i, sel_f = pl.pallas_call(
        _select_body,
        out_shape=[jax.ShapeDtypeStruct((16,), jnp.int32),
                   jax.ShapeDtypeStruct((16,), jnp.float32)],
        in_specs=[pl.BlockSpec(memory_space=pltpu.SMEM),
                  pl.BlockSpec(memory_space=pltpu.VMEM)],
        out_specs=[pl.BlockSpec(memory_space=pltpu.SMEM),
                   pl.BlockSpec(memory_space=pltpu.SMEM)],
    )(temperature, advS)

    mesh = plsc.VectorSubcoreMesh(core_axis_name="c", subcore_axis_name="s")
    sc = functools.partial(
        pl.kernel,
        mesh=mesh,
        out_type=jax.ShapeDtypeStruct((NW, 2, 16), jnp.float32),
        scratch_types=[
            pltpu.VMEM((CR, A), jnp.float32),
            pltpu.VMEM((CR, A), jnp.float32),
            pltpu.VMEM((CR, A), jnp.float32),
            pltpu.VMEM((CR // 16, 16), jnp.float32),
            pltpu.VMEM((16,), jnp.int32),
            pltpu.VMEM((16,), jnp.float32),
            pltpu.VMEM((2, 16), jnp.float32),
        ],
    )(_sc_body)
    parts = sc(action_mean, action_std, actions, advG, sel_i, sel_f)

    d_tot = jnp.sum(parts[:, 0, :])
    nu_tot = jnp.sum(parts[:, 1, :])
    return -(nu_tot / d_tot + jnp.float32(_LOGC))


# speedup vs baseline: 1.5144x; 1.5144x over previous
import functools
import math

import jax
import jax.numpy as jnp
from jax import lax
from jax.experimental import pallas as pl
from jax.experimental.pallas import tpu as pltpu
from jax.experimental.pallas import tpu_sc as plsc

N = 262144
A = 16
K = N // 2
SEL_COLS = 2048

NW = 32
TPW = N // NW
CR = 2048
NCH = TPW // CR
NG = CR // 16

_LOGC = -0.5 * A * math.log(2.0 * math.pi)


def _sortable_i32(x_f32):
    b = lax.bitcast_convert_type(x_f32, jnp.int32)
    return b ^ ((b >> 31) & jnp.int32(0x7FFFFFFF))


def _select_body(temp_ref, adv_ref, out_i_ref, out_f_ref):
    a = adv_ref[...]
    s = _sortable_i32(a)

    def vbody(t, cand):
        trial = cand + (jnp.int32(1) << (31 - t))
        c = jnp.sum((s >= trial).astype(jnp.int32))
        return lax.select(c >= K, trial, cand)

    theta = lax.fori_loop(0, 32, vbody, jnp.int32(-2147483648))

    c_gt = jnp.sum((s > theta).astype(jnp.int32))
    t_need = K - c_gt

    eq = (s == theta)
    idx = (lax.broadcasted_iota(jnp.int32, (N // SEL_COLS, SEL_COLS), 0)
           * SEL_COLS
           + lax.broadcasted_iota(jnp.int32, (N // SEL_COLS, SEL_COLS), 1))

    def ibody(t, m):
        trial = m | (jnp.int32(1) << (17 - t))
        c = jnp.sum((eq & (idx < trial)).astype(jnp.int32))
        return lax.select(c < t_need, trial, m)

    mbound = lax.fori_loop(0, 18, ibody, jnp.int32(0))

    out_i_ref[0] = theta
    out_i_ref[1] = mbound
    for t in range(2, 16):
        out_i_ref[t] = jnp.int32(0)
    out_f_ref[0] = jnp.max(a)
    out_f_ref[1] = temp_ref[0] + jnp.float32(0.001)
    for t in range(2, 16):
        out_f_ref[t] = jnp.float32(0.0)


def _sc_body(m_hbm, s_hbm, a_hbm, advg_hbm, pi_hbm, pf_hbm, out_hbm,
             mbuf, sbuf, abuf, advbuf, pibuf, pfbuf, obuf):
    nc = 2
    wid = lax.axis_index("s") * nc + lax.axis_index("c")
    base = wid * TPW

    pltpu.sync_copy(pi_hbm, pibuf)
    pltpu.sync_copy(pf_hbm, pfbuf)
    pivec = pibuf[...]
    pfvec = pfbuf[...]
    theta = pivec[0]
    mbound = pivec[1]
    mx = pfvec[0]
    invtpv = jnp.full((16,), 1.0, jnp.float32) / pfvec
    invtp = invtpv[1]

    half = jnp.float32(-0.5)
    c1 = jnp.float32(2.0)
    c3 = jnp.float32(2.0 / 3.0)
    c5 = jnp.float32(0.4)
    c7 = jnp.float32(2.0 / 7.0)

    def chunk_body(c, carry):
        nu_v, d_v = carry
        cb = base + c * CR
        cbt = pl.multiple_of(cb * A, CR * A)
        cba = pl.multiple_of(cb, CR)
        pltpu.sync_copy(m_hbm.at[pl.ds(cbt, CR * A)], mbuf)
        pltpu.sync_copy(s_hbm.at[pl.ds(cbt, CR * A)], sbuf)
        pltpu.sync_copy(a_hbm.at[pl.ds(cbt, CR * A)], abuf)
        pltpu.sync_copy(advg_hbm.at[pl.ds(cba, CR)], advbuf)

        def group_body(g, carry2):
            nu_v2, d_v2 = carry2
            adv_v = advbuf[pl.ds(g * 16, 16)]
            sv = _sortable_i32(adv_v)
            idx_v = cb + g * 16 + lax.iota(jnp.int32, 16)
            sel = (sv > theta) | ((sv == theta) & (idx_v <= mbound))
            w_v = jnp.where(sel, jnp.exp((adv_v - mx) * invtp),
                            jnp.float32(0.0))
            d_v2 = d_v2 + w_v
            gb = g * (16 * A)
            for j in range(16):
                mv = mbuf[pl.ds(gb + j * A, A)]
                stdv = sbuf[pl.ds(gb + j * A, A)]
                av = abuf[pl.ds(gb + j * A, A)]
                rs = jnp.full((16,), 1.0, jnp.float32) / stdv
                t = (av - mv) * rs
                wl = (rs - jnp.float32(1.0)) / (rs + jnp.float32(1.0))
                wl2 = wl * wl
                logs = -(wl * (c1 + wl2 * (c3 + wl2 * (c5 + wl2 * c7))))
                term = half * (t * t) - logs
                nu_v2 = nu_v2 + w_v[j] * term
            return (nu_v2, d_v2)

        return lax.fori_loop(0, NG, group_body, (nu_v, d_v))

    zero = jnp.zeros((16,), jnp.float32)
    nu_v, d_v = lax.fori_loop(0, NCH, chunk_body, (zero, zero))

    obuf[0] = d_v
    obuf[1] = nu_v
    pltpu.sync_copy(obuf, out_hbm.at[wid])


@jax.jit
def kernel(action_mean, action_std, actions, temperature, advantages):
    advS = advantages.reshape(N // SEL_COLS, SEL_COLS)
    mean_f = action_mean.reshape(N * A)
    std_f = action_std.reshape(N * A)
    act_f = actions.reshape(N * A)

    sel_i, sel_f = pl.pallas_call(
        _select_body,
        out_shape=[jax.ShapeDtypeStruct((16,), jnp.int32),
                   jax.ShapeDtypeStruct((16,), jnp.float32)],
        in_specs=[pl.BlockSpec(memory_space=pltpu.SMEM),
                  pl.BlockSpec(memory_space=pltpu.VMEM)],
        out_specs=[pl.BlockSpec(memory_space=pltpu.SMEM),
                   pl.BlockSpec(memory_space=pltpu.SMEM)],
    )(temperature, advS)

    mesh = plsc.VectorSubcoreMesh(core_axis_name="c", subcore_axis_name="s")
    sc = functools.partial(
        pl.kernel,
        mesh=mesh,
        out_type=jax.ShapeDtypeStruct((NW, 2, 16), jnp.float32),
        scratch_types=[
            pltpu.VMEM((CR * A,), jnp.float32),
            pltpu.VMEM((CR * A,), jnp.float32),
            pltpu.VMEM((CR * A,), jnp.float32),
            pltpu.VMEM((CR,), jnp.float32),
            pltpu.VMEM((16,), jnp.int32),
            pltpu.VMEM((16,), jnp.float32),
            pltpu.VMEM((2, 16), jnp.float32),
        ],
    )(_sc_body)
    parts = sc(mean_f, std_f, act_f, advantages, sel_i, sel_f)

    d_tot = jnp.sum(parts[:, 0, :])
    nu_tot = jnp.sum(parts[:, 1, :])
    return -(nu_tot / d_tot + jnp.float32(_LOGC))


# speedup vs baseline: 1.5181x; 1.0025x over previous
import functools
import math

import jax
import jax.numpy as jnp
from jax import lax
from jax.experimental import pallas as pl
from jax.experimental.pallas import tpu as pltpu
from jax.experimental.pallas import tpu_sc as plsc

N = 262144
A = 16
K = N // 2
SEL_COLS = 2048

NW = 32
TPW = N // NW
CR = 2048
NCH = TPW // CR
NG = CR // 16

_LOGC = -0.5 * A * math.log(2.0 * math.pi)


def _sortable_i32(x_f32):
    b = lax.bitcast_convert_type(x_f32, jnp.int32)
    return b ^ ((b >> 31) & jnp.int32(0x7FFFFFFF))


def _select_body(temp_ref, adv_ref, out_i_ref, out_f_ref):
    a = adv_ref[...]
    s = _sortable_i32(a)

    def vbody(t, cand):
        trial = cand + (jnp.int32(1) << (31 - t))
        c = jnp.sum((s >= trial).astype(jnp.int32))
        return lax.select(c >= K, trial, cand)

    theta = lax.fori_loop(0, 32, vbody, jnp.int32(-2147483648))

    c_gt = jnp.sum((s > theta).astype(jnp.int32))
    t_need = K - c_gt

    eq = (s == theta)
    idx = (lax.broadcasted_iota(jnp.int32, (N // SEL_COLS, SEL_COLS), 0)
           * SEL_COLS
           + lax.broadcasted_iota(jnp.int32, (N // SEL_COLS, SEL_COLS), 1))

    def ibody(t, m):
        trial = m | (jnp.int32(1) << (17 - t))
        c = jnp.sum((eq & (idx < trial)).astype(jnp.int32))
        return lax.select(c < t_need, trial, m)

    mbound = lax.fori_loop(0, 18, ibody, jnp.int32(0))

    out_i_ref[0] = theta
    out_i_ref[1] = mbound
    for t in range(2, 16):
        out_i_ref[t] = jnp.int32(0)
    out_f_ref[0] = jnp.max(a)
    out_f_ref[1] = temp_ref[0] + jnp.float32(0.001)
    for t in range(2, 16):
        out_f_ref[t] = jnp.float32(0.0)


def _sc_body(m_hbm, s_hbm, a_hbm, advg_hbm, pi_hbm, pf_hbm, out_hbm,
             mbuf, sbuf, abuf, advbuf, pibuf, pfbuf, obuf):
    nc = 2
    wid = lax.axis_index("s") * nc + lax.axis_index("c")
    base = wid * TPW

    pltpu.sync_copy(pi_hbm, pibuf)
    pltpu.sync_copy(pf_hbm, pfbuf)
    pivec = pibuf[...]
    pfvec = pfbuf[...]
    theta = pivec[0]
    mbound = pivec[1]
    mx = pfvec[0]
    invtpv = jnp.full((16,), 1.0, jnp.float32) / pfvec
    invtp = invtpv[1]

    half = jnp.float32(-0.5)
    c1 = jnp.float32(2.0)
    c3 = jnp.float32(2.0 / 3.0)
    c5 = jnp.float32(0.4)
    c7 = jnp.float32(2.0 / 7.0)

    def chunk_body(c, carry):
        nu_v = carry[:4]
        d_v = carry[4]
        cb = base + c * CR
        cbt = pl.multiple_of(cb * A, CR * A)
        cba = pl.multiple_of(cb, CR)
        pltpu.sync_copy(m_hbm.at[pl.ds(cbt, CR * A)], mbuf)
        pltpu.sync_copy(s_hbm.at[pl.ds(cbt, CR * A)], sbuf)
        pltpu.sync_copy(a_hbm.at[pl.ds(cbt, CR * A)], abuf)
        pltpu.sync_copy(advg_hbm.at[pl.ds(cba, CR)], advbuf)

        def group_body(g, carry2):
            nu0, nu1, nu2, nu3, d_v2 = carry2
            adv_v = advbuf[pl.ds(g * 16, 16)]
            sv = _sortable_i32(adv_v)
            idx_v = cb + g * 16 + lax.iota(jnp.int32, 16)
            sel = (sv > theta) | ((sv == theta) & (idx_v <= mbound))
            w_v = jnp.where(sel, jnp.exp((adv_v - mx) * invtp),
                            jnp.float32(0.0))
            d_v2 = d_v2 + w_v
            gb = g * (16 * A)
            acc = [nu0, nu1, nu2, nu3]
            one = jnp.float32(1.0)
            for j in range(16):
                mv = mbuf[pl.ds(gb + j * A, A)]
                stdv = sbuf[pl.ds(gb + j * A, A)]
                av = abuf[pl.ds(gb + j * A, A)]
                q = jnp.full((16,), 1.0, jnp.float32) / (stdv + stdv * stdv)
                rs = q + q * stdv
                t = (av - mv) * rs
                wl = (one - stdv) * (q * stdv)
                wl2 = wl * wl
                logs = -(wl * (c1 + wl2 * (c3 + wl2 * (c5 + wl2 * c7))))
                term = half * (t * t) - logs
                acc[j % 4] = acc[j % 4] + w_v[j] * term
            return (acc[0], acc[1], acc[2], acc[3], d_v2)

        return lax.fori_loop(0, NG, group_body,
                             (nu_v[0], nu_v[1], nu_v[2], nu_v[3], d_v))

    zero = jnp.zeros((16,), jnp.float32)
    fin = lax.fori_loop(0, NCH, chunk_body,
                        (zero, zero, zero, zero, zero))

    obuf[0] = fin[4]
    obuf[1] = ((fin[0] + fin[1]) + (fin[2] + fin[3]))
    pltpu.sync_copy(obuf, out_hbm.at[wid])


@jax.jit
def kernel(action_mean, action_std, actions, temperature, advantages):
    advS = advantages.reshape(N // SEL_COLS, SEL_COLS)
    mean_f = action_mean.reshape(N * A)
    std_f = action_std.reshape(N * A)
    act_f = actions.reshape(N * A)

    sel_i, sel_f = pl.pallas_call(
        _select_body,
        out_shape=[jax.ShapeDtypeStruct((16,), jnp.int32),
                   jax.ShapeDtypeStruct((16,), jnp.float32)],
        in_specs=[pl.BlockSpec(memory_space=pltpu.SMEM),
                  pl.BlockSpec(memory_space=pltpu.VMEM)],
        out_specs=[pl.BlockSpec(memory_space=pltpu.SMEM),
                   pl.BlockSpec(memory_space=pltpu.SMEM)],
    )(temperature, advS)

    mesh = plsc.VectorSubcoreMesh(core_axis_name="c", subcore_axis_name="s")
    sc = functools.partial(
        pl.kernel,
        mesh=mesh,
        out_type=jax.ShapeDtypeStruct((NW, 2, 16), jnp.float32),
        scratch_types=[
            pltpu.VMEM((CR * A,), jnp.float32),
            pltpu.VMEM((CR * A,), jnp.float32),
            pltpu.VMEM((CR * A,), jnp.float32),
            pltpu.VMEM((CR,), jnp.float32),
            pltpu.VMEM((16,), jnp.int32),
            pltpu.VMEM((16,), jnp.float32),
            pltpu.VMEM((2, 16), jnp.float32),
        ],
    )(_sc_body)
    parts = sc(mean_f, std_f, act_f, advantages, sel_i, sel_f)

    d_tot = jnp.sum(parts[:, 0, :])
    nu_tot = jnp.sum(parts[:, 1, :])
    return -(nu_tot / d_tot + jnp.float32(_LOGC))
